# baseline (speedup 1.0000x reference)
import jax
import jax.numpy as jnp
from jax.experimental import pallas as pl
from jax.experimental.pallas import tpu as pltpu

N = 512
D = 128
ED = 16
MD = 64
T = 3
BI = 128
BP = 256
NI = N // BI
NJ = N // (2 * BP)
NP = N // 2
KC = 2 * ED + 2
BIG = 1e9


def _mpnn_kernel(X_ref, Epk_ref, deg_ref,
                 W1iT_ref, W1jT_ref, Wcat_ref, b1_ref,
                 W2bd_ref, b2_ref, WihT_ref, WhhT_ref, bih_ref, bhh_ref,
                 out_ref, Hc, Pi2, Pj2, S2):
    t = pl.program_id(0)
    i = pl.program_id(1)
    j = pl.program_id(2)

    @pl.when(jnp.logical_and(i == 0, j == 0))
    def _step_start():
        @pl.when(t == 0)
        def _():
            Hc[...] = X_ref[...]

        @pl.when(t > 0)
        def _():
            Hc[...] = out_ref[...]

        H = Hc[...]
        pi = jnp.dot(H, W1iT_ref[...],
                     preferred_element_type=jnp.float32) + b1_ref[...]
        Pi2[...] = jnp.concatenate([pi, pi], axis=1)
        pj = jnp.dot(H, W1jT_ref[...], preferred_element_type=jnp.float32)
        Pj2[...] = jnp.concatenate([pj[:NP, :], pj[NP:, :]], axis=1)

    @pl.when(j == 0)
    def _row_start():
        S2[...] = jnp.zeros_like(S2)

    Eb = Epk_ref[...]
    pe = jax.lax.dot_general(Eb, Wcat_ref[...],
                             (((2,), (0,)), ((), ())),
                             preferred_element_type=jnp.float32)
    pib = Pi2[pl.ds(i * BI, BI), :]
    pjb = Pj2[pl.ds(j * BP, BP), :]
    pre = pe + pib[:, None, :] + pjb[None, :, :]
    r = jnp.maximum(pre, 0.0)
    m2 = jax.lax.dot_general(r, W2bd_ref[...],
                             (((2,), (0,)), ((), ())),
                             preferred_element_type=jnp.float32)
    S2[...] += jnp.sum(m2, axis=1)

    @pl.when(j == NJ - 1)
    def _finish_row():
        Sb = S2[...]
        S = Sb[:, :MD] + Sb[:, MD:]
        M = S + deg_ref[pl.ds(i * BI, BI), :] * b2_ref[...]
        Hi = Hc[pl.ds(i * BI, BI), :]
        gi = jnp.dot(M, WihT_ref[...],
                     preferred_element_type=jnp.float32) + bih_ref[...]
        gh = jnp.dot(Hi, WhhT_ref[...],
                     preferred_element_type=jnp.float32) + bhh_ref[...]
        rg = jax.nn.sigmoid(gi[:, :D] + gh[:, :D])
        z = jax.nn.sigmoid(gi[:, D:2 * D] + gh[:, D:2 * D])
        n = jnp.tanh(gi[:, 2 * D:] + rg * gh[:, 2 * D:])
        newH = (1.0 - z) * n + z * Hi
        out_ref[pl.ds(i * BI, BI), :] = newH


@jax.jit
def kernel(X, A, E, W1, b1, W2, b2, Wih, Whh, bih, bhh):
    W1iT = W1[:, :D].T
    W1jT = W1[:, D:2 * D].T
    W1eT = W1[:, 2 * D:].T
    Wcat = jnp.zeros((KC, 2 * MD), jnp.float32)
    Wcat = Wcat.at[:ED, :MD].set(W1eT)
    Wcat = Wcat.at[ED, :MD].set(BIG)
    Wcat = Wcat.at[ED + 1:2 * ED + 1, MD:].set(W1eT)
    Wcat = Wcat.at[2 * ED + 1, MD:].set(BIG)
    b1r = b1.reshape(1, MD)
    W2T = W2.T
    W2bd = jnp.zeros((2 * MD, 2 * MD), jnp.float32)
    W2bd = W2bd.at[:MD, :MD].set(W2T).at[MD:, MD:].set(W2T)
    b2r = b2.reshape(1, MD)
    WihT = Wih.T
    WhhT = Whh.T
    bihr = bih.reshape(1, 3 * D)
    bhhr = bhh.reshape(1, 3 * D)
    mask = (A > 0.0).astype(jnp.float32)
    F = mask - 1.0
    Epk = jnp.concatenate(
        [E[:, :NP, :], F[:, :NP, None], E[:, NP:, :], F[:, NP:, None]],
        axis=2)
    degr = mask.sum(axis=1, keepdims=True)

    grid = (T, NI, NJ)
    full = lambda t, i, j: (0, 0)
    out = pl.pallas_call(
        _mpnn_kernel,
        grid=grid,
        in_specs=[
            pl.BlockSpec((N, D), full),
            pl.BlockSpec((BI, BP, KC), lambda t, i, j: (i, j, 0)),
            pl.BlockSpec((N, 1), lambda t, i, j: (0, 0)),
            pl.BlockSpec((D, MD), full),
            pl.BlockSpec((D, MD), full),
            pl.BlockSpec((KC, 2 * MD), full),
            pl.BlockSpec((1, MD), full),
            pl.BlockSpec((2 * MD, 2 * MD), full),
            pl.BlockSpec((1, MD), full),
            pl.BlockSpec((MD, 3 * D), full),
            pl.BlockSpec((D, 3 * D), full),
            pl.BlockSpec((1, 3 * D), full),
            pl.BlockSpec((1, 3 * D), full),
        ],
        out_specs=pl.BlockSpec((N, D), full),
        out_shape=jax.ShapeDtypeStruct((N, D), jnp.float32),
        scratch_shapes=[
            pltpu.VMEM((N, D), jnp.float32),
            pltpu.VMEM((N, 2 * MD), jnp.float32),
            pltpu.VMEM((NP, 2 * MD), jnp.float32),
            pltpu.VMEM((BI, 2 * MD), jnp.float32),
        ],
        compiler_params=pltpu.CompilerParams(
            dimension_semantics=("arbitrary", "arbitrary", "arbitrary")),
    )(X, Epk, degr, W1iT, W1jT, Wcat, b1r, W2bd, b2r,
      WihT, WhhT, bihr, bhhr)
    return out
